# Initial kernel scaffold; baseline (speedup 1.0000x reference)
import jax
import jax.numpy as jnp
from jax.experimental import pallas as pl


def kernel(x, token_emb, pos_emb):
    raise NotImplementedError("write your pallas kernel here")


# speedup vs baseline: 2.3159x; 2.3159x over previous
import functools

import jax
import jax.numpy as jnp
from jax import lax
from jax.experimental import pallas as pl
from jax.experimental.pallas import tpu as pltpu
from jax.experimental.pallas import tpu_sc as plsc

_D = 64
_B = 1024
_S = 200
_CHUNK = 100
_NCHUNKS = (_B * _S) // _CHUNK
_NW = 32
_CPW = _NCHUNKS // _NW
_LANES = 16
_VPR = _D // _LANES


def _sc_embed(x2d, table, pos2d):
    mesh = plsc.VectorSubcoreMesh(core_axis_name="c", subcore_axis_name="s")

    @functools.partial(
        pl.kernel,
        mesh=mesh,
        out_type=jax.ShapeDtypeStruct((_NCHUNKS, _CHUNK, _D), jnp.float32),
        scratch_types=[
            pltpu.VMEM((_S, _D), jnp.float32),
            pltpu.VMEM((_CPW, _CHUNK), jnp.int32),
            pltpu.VMEM((_CHUNK, _D), jnp.float32),
            pltpu.VMEM((_CHUNK, _D), jnp.float32),
            pltpu.SemaphoreType.DMA,
            pltpu.SemaphoreType.DMA,
        ],
        compiler_params=pltpu.CompilerParams(use_tc_tiling_on_sc=False),
    )
    def k(x_hbm, tab_hbm, pos_hbm, out_hbm, pos_v, idx_v, buf0, buf1,
          sem0, sem1):
        wid = lax.axis_index("s") * 2 + lax.axis_index("c")
        c_base = wid * _CPW

        pltpu.sync_copy(pos_hbm, pos_v)
        pltpu.sync_copy(x_hbm.at[pl.ds(c_base, _CPW)], idx_v)

        def gather_start(c, buf, sem):
            pltpu.make_async_copy(tab_hbm.at[idx_v.at[c]], buf, sem).start()

        def gather_wait(buf, sem):
            pltpu.make_async_copy(tab_hbm.at[idx_v.at[0]], buf, sem).wait()

        gather_start(0, buf0, sem0)
        gather_start(1, buf1, sem1)

        def add_pos(buf, sbase):
            def rbody(r, _):
                for i in range(_VPR):
                    sl = pl.ds(_LANES * i, _LANES)
                    buf[r, sl] = buf[r, sl] + pos_v[sbase + r, sl]
                return 0
            lax.fori_loop(0, _CHUNK, rbody, 0, unroll=2)

        def step(t, _):
            gather_wait(buf0, sem0)
            add_pos(buf0, 0)
            pltpu.sync_copy(buf0, out_hbm.at[c_base + 2 * t])

            @pl.when(t < _CPW // 2 - 1)
            def _():
                gather_start(2 * t + 2, buf0, sem0)

            gather_wait(buf1, sem1)
            add_pos(buf1, _CHUNK)
            pltpu.sync_copy(buf1, out_hbm.at[c_base + 2 * t + 1])

            @pl.when(t < _CPW // 2 - 1)
            def _():
                gather_start(2 * t + 3, buf1, sem1)

            return 0

        lax.fori_loop(0, _CPW // 2, step, 0)

    return k(x2d, table, pos2d)


def kernel(x, token_emb, pos_emb):
    seq = x.shape[1]
    x2d = x.reshape(_NCHUNKS, _CHUNK).astype(jnp.int32)
    pos2d = pos_emb[0, :seq, :].astype(jnp.float32)
    out = _sc_embed(x2d, token_emb, pos2d)
    return out.reshape(_B, _S, _D)


# speedup vs baseline: 2.3361x; 1.0087x over previous
import functools

import jax
import jax.numpy as jnp
from jax import lax
from jax.experimental import pallas as pl
from jax.experimental.pallas import tpu as pltpu
from jax.experimental.pallas import tpu_sc as plsc

_D = 64
_B = 1024
_S = 200
_CHUNK = 100
_NCHUNKS = (_B * _S) // _CHUNK
_NW = 32
_CPW = _NCHUNKS // _NW
_BPW = _CPW // 2
_LANES = 16
_VPR = _D // _LANES
_NBUF = 4


def _sc_embed(x2d, table, pos2d):
    mesh = plsc.VectorSubcoreMesh(core_axis_name="c", subcore_axis_name="s")

    @functools.partial(
        pl.kernel,
        mesh=mesh,
        out_type=jax.ShapeDtypeStruct((_B, _S, _D), jnp.float32),
        scratch_types=[
            pltpu.VMEM((_S, _D), jnp.float32),
            pltpu.VMEM((_CPW, _CHUNK), jnp.int32),
            [pltpu.VMEM((_CHUNK, _D), jnp.float32)] * _NBUF,
            [pltpu.VMEM((_CHUNK, _D), jnp.float32)] * _NBUF,
            [pltpu.SemaphoreType.DMA] * _NBUF,
            [pltpu.SemaphoreType.DMA] * _NBUF,
        ],
        compiler_params=pltpu.CompilerParams(use_tc_tiling_on_sc=False),
    )
    def k(x_hbm, tab_hbm, pos_hbm, out_hbm, pos_v, idx_v, gbufs, obufs,
          gsems, wsems):
        wid = lax.axis_index("s") * 2 + lax.axis_index("c")
        c_base = wid * _CPW
        b_base = wid * _BPW

        pltpu.sync_copy(pos_hbm, pos_v)
        pltpu.sync_copy(x_hbm.at[pl.ds(c_base, _CPW)], idx_v)

        def gather_start(c, buf, sem):
            pltpu.make_async_copy(tab_hbm.at[idx_v.at[c]], buf, sem).start()

        def gather_wait(buf, sem):
            pltpu.make_async_copy(tab_hbm.at[idx_v.at[0]], buf, sem).wait()

        def write_dst(b, half):
            return out_hbm.at[b, pl.ds(half * _CHUNK, _CHUNK)]

        def write_wait(sem):
            pltpu.make_async_copy(write_dst(0, 0), write_dst(0, 0), sem).wait()

        for j in range(_NBUF):
            gather_start(j, gbufs[j], gsems[j])

        def add_pos(src, dst, sbase):
            def rbody(r, _):
                for i in range(_VPR):
                    sl = pl.ds(_LANES * i, _LANES)
                    dst[r, sl] = src[r, sl] + pos_v[sbase + r, sl]
                return 0
            lax.fori_loop(0, _CHUNK, rbody, 0, unroll=4)

        def step(t, _):
            for j in range(_NBUF):
                half = j & 1
                b = b_base + 2 * t + (j >> 1)
                gather_wait(gbufs[j], gsems[j])

                @pl.when(t > 0)
                def _():
                    write_wait(wsems[j])

                add_pos(gbufs[j], obufs[j], half * _CHUNK)
                pltpu.make_async_copy(
                    obufs[j], write_dst(b, half), wsems[j]).start()

                @pl.when(t < _CPW // _NBUF - 1)
                def _():
                    gather_start(_NBUF * t + j + _NBUF, gbufs[j], gsems[j])

            return 0

        lax.fori_loop(0, _CPW // _NBUF, step, 0)

        for j in range(_NBUF):
            write_wait(wsems[j])

    return k(x2d, table, pos2d)


def kernel(x, token_emb, pos_emb):
    seq = x.shape[1]
    x2d = x.reshape(_NCHUNKS, _CHUNK).astype(jnp.int32)
    pos2d = pos_emb[0, :seq, :].astype(jnp.float32)
    return _sc_embed(x2d, token_emb, pos2d)
